# baseline (speedup 1.0000x reference)
import functools

import jax
import jax.numpy as jnp
from jax import lax
from jax.experimental import pallas as pl
from jax.experimental.pallas import tpu as pltpu
from jax.experimental.pallas import tpu_sc as plsc

N = 10000
NP = 10240
D = 128
E = 320000
NC, NS = 2, 16
CHT = 160
EP = CHT * NS * 128
ER = EP // 128
RPT = NP // NS
DUMMY = N

def _sc_prep_body(xpad, perm2, edges2, ones128, zrows,
                  x2_out, outdeg_out, indeg_out,
                  degacc, idx_v, ones_v, pidx_v, xbuf, sd):
    c = lax.axis_index("c")
    s = lax.axis_index("s")
    w = c * NS + s

    pltpu.sync_copy(zrows, degacc.at[pl.ds(s * RPT, RPT)])
    plsc.subcore_barrier()

    pltpu.sync_copy(ones128, ones_v)

    def deg_batch(b, carry):
        pltpu.sync_copy(edges2.at[c, pl.ds(s * CHT + b * IB, IB)], idx_v)

        def fire(j, carry2):
            pltpu.async_copy(ones_v, degacc.at[idx_v.at[j]], sd, add=True)
            return carry2
        lax.fori_loop(0, IB, fire, 0)

        def drain(j, carry2):
            pltpu.make_async_copy(ones_v, degacc.at[idx_v.at[0]], sd).wait()
            return carry2
        return lax.fori_loop(0, IB, drain, carry)
    lax.fori_loop(0, NB, deg_batch, 0)

    pltpu.sync_copy(xpad.at[pl.ds(w * (NP // 32), NP // 32)],
                    x2_out.at[pl.ds(w * (NP // 32), NP // 32)])
    pltpu.sync_copy(perm2.at[w], pidx_v)

    def xp_body(k, carry):
        chunk = w * 5 + k
        pltpu.sync_copy(xpad.at[pidx_v.at[k]], xbuf)
        pltpu.sync_copy(xbuf, x2_out.at[pl.ds(NP + chunk * 64, 64)])
        return carry
    lax.fori_loop(0, 5, xp_body, 0)

    plsc.subcore_barrier()

    @pl.when(c == 0)
    def _():
        pltpu.sync_copy(degacc.at[pl.ds(s * RPT, RPT)],
                        outdeg_out.at[pl.ds(s * RPT, RPT)])

    @pl.when(c == 1)
    def _():
        pltpu.sync_copy(degacc.at[pl.ds(s * RPT, RPT)],
                        indeg_out.at[pl.ds(s * RPT, RPT)])


@functools.cache
def _get_sc_prep():
    mesh = plsc.VectorSubcoreMesh(
        core_axis_name="c", subcore_axis_name="s",
        num_cores=NC, num_subcores=NS)
    return pl.kernel(
        _sc_prep_body,
        out_type=[
            jax.ShapeDtypeStruct((2 * NP, D), jnp.float32),
            jax.ShapeDtypeStruct((NP, D), jnp.float32),
            jax.ShapeDtypeStruct((NP, D), jnp.float32),
        ],
        mesh=mesh,
        scratch_types=[
            pltpu.VMEM_SHARED((NP, D), jnp.float32),
            pltpu.VMEM((IB, 128), jnp.int32),
            pltpu.VMEM((128, D), jnp.float32),
            pltpu.VMEM((5, 64), jnp.int32),
            pltpu.VMEM((64, D), jnp.float32),
            pltpu.SemaphoreType.DMA,
        ],
    )


IB = 8
NB = CHT // IB


def _sc_scatter_body(table, src3, dst2, zrows, agg_out,
                     acc, src_v, dst_v, buf0, buf1,
                     g0, g1, g2, g3, g4, g5, g6, g7,
                     g8, g9, g10, g11, g12, g13, g14, g15, ss0, ss1):
    sgs0 = [g0, g1, g2, g3, g4, g5, g6, g7]
    sgs1 = [g8, g9, g10, g11, g12, g13, g14, g15]
    c = lax.axis_index("c")
    s = lax.axis_index("s")

    pltpu.sync_copy(zrows, acc.at[pl.ds(s * RPT, RPT)])
    plsc.subcore_barrier()

    def gath(g, q, buf, sem):
        pltpu.async_copy(table.at[src_v.at[g]],
                         buf.at[pl.ds(q * 16, 16)], sem)

    def wait_gath(q, buf, sem):
        pltpu.make_async_copy(table.at[src_v.at[0]],
                              buf.at[pl.ds(q * 16, 16)], sem).wait()

    def scat(m, buf, sem):
        pltpu.async_copy(buf, acc.at[dst_v.at[m]], sem, add=True)

    def wait_scat(buf, sem):
        pltpu.make_async_copy(buf, acc.at[dst_v.at[0]], sem).wait()

    def batch(b, carry):
        pltpu.sync_copy(src3.at[c, pl.ds(s * 8 * CHT + b * 8 * IB, 8 * IB)],
                        src_v)
        pltpu.sync_copy(dst2.at[pl.ds(s * CHT + b * IB, IB)], dst_v)
        for q in range(8):
            gath(q, q, buf0, sgs0[q])
            gath(8 + q, q, buf1, sgs1[q])

        def body(k, carry2):
            m = 2 * k
            for q in range(8):
                wait_gath(q, buf0, sgs0[q])
            scat(m, buf0, ss0)
            for q in range(8):
                wait_gath(q, buf1, sgs1[q])
            scat(m + 1, buf1, ss1)

            @pl.when(m + 2 < IB)
            def _():
                wait_scat(buf0, ss0)
                for q in range(8):
                    gath(8 * (m + 2) + q, q, buf0, sgs0[q])

            @pl.when(m + 3 < IB)
            def _():
                wait_scat(buf1, ss1)
                for q in range(8):
                    gath(8 * (m + 3) + q, q, buf1, sgs1[q])
            return carry2
        r = lax.fori_loop(0, IB // 2, body, carry)
        wait_scat(buf0, ss0)
        wait_scat(buf1, ss1)
        return r
    lax.fori_loop(0, NB, batch, 0)

    plsc.subcore_barrier()
    pltpu.sync_copy(acc.at[pl.ds(s * RPT, RPT)],
                    agg_out.at[pl.ds(c * NP + s * RPT, RPT)])


@functools.cache
def _get_sc_scatter():
    mesh = plsc.VectorSubcoreMesh(
        core_axis_name="c", subcore_axis_name="s",
        num_cores=NC, num_subcores=NS)
    return pl.kernel(
        _sc_scatter_body,
        out_type=jax.ShapeDtypeStruct((2 * NP, D), jnp.float32),
        mesh=mesh,
        scratch_types=[
            pltpu.VMEM_SHARED((NP, D), jnp.float32),
            pltpu.VMEM((8 * IB, 16), jnp.int32),
            pltpu.VMEM((IB, 128), jnp.int32),
            pltpu.VMEM((128, D), jnp.float32),
            pltpu.VMEM((128, D), jnp.float32),
        ] + [pltpu.SemaphoreType.DMA] * 18,
    )


BM = 256
GRID = (2 * NP) // BM


def _tc_stage1(x_ref, od_ref, w_ref, o_ref):
    ns = lax.rsqrt(jnp.maximum(od_ref[:, 0:1], 1.0))
    o_ref[...] = jnp.dot(x_ref[...] * ns, w_ref[...],
                         preferred_element_type=jnp.float32)


def _tc_stage2(agg_ref, id_ref, od_ref, b_ref, a_ref, w_ref, o_ref):
    nd = lax.rsqrt(jnp.maximum(id_ref[:, 0:1], 1.0))
    t = agg_ref[...] * nd + b_ref[...]
    p = jnp.maximum(t, 0.0) + a_ref[...] * jnp.minimum(t, 0.0)
    ns = lax.rsqrt(jnp.maximum(od_ref[:, 0:1], 1.0))
    o_ref[...] = jnp.dot(p * ns, w_ref[...],
                         preferred_element_type=jnp.float32)


BM3 = 2048


def _tc_stage3(agg_ref, id_ref, b_ref, a_ref, mwt_ref, mb_ref, o_ref):
    nd = lax.rsqrt(jnp.maximum(id_ref[:, 0:1], 1.0))
    t = agg_ref[...] * nd + b_ref[...]
    p = jnp.maximum(t, 0.0) + a_ref[...] * jnp.minimum(t, 0.0)
    q = jnp.dot(p, mwt_ref[...], preferred_element_type=jnp.float32)
    q = q + mb_ref[...]
    o_ref[...] = jnp.sum(q, axis=1).reshape(BM3 // BM, BM)


def _row_spec():
    return pl.BlockSpec((BM, D), lambda i: (i, 0))


def _deg_spec():
    return pl.BlockSpec((BM, D), lambda i: (i % (NP // BM), 0))


def _full(shape):
    return pl.BlockSpec(shape, lambda i: tuple(0 for _ in shape))


def _stage1(x2, outdeg, W1):
    return pl.pallas_call(
        _tc_stage1, grid=(GRID,),
        in_specs=[_row_spec(), _deg_spec(), _full((D, D))],
        out_specs=_row_spec(),
        out_shape=jax.ShapeDtypeStruct((2 * NP, D), jnp.float32),
    )(x2, outdeg, W1)


def _stage2(agg, indeg, outdeg, b, a, W2):
    return pl.pallas_call(
        _tc_stage2, grid=(GRID,),
        in_specs=[_row_spec(), _deg_spec(), _deg_spec(),
                  _full((1, D)), _full((1, D)), _full((D, D))],
        out_specs=_row_spec(),
        out_shape=jax.ShapeDtypeStruct((2 * NP, D), jnp.float32),
    )(agg, indeg, outdeg, b, a, W2)


def _stage3(agg, indeg, b, a, mW, mb):
    grid3 = (2 * NP) // BM3
    out = pl.pallas_call(
        _tc_stage3, grid=(grid3,),
        in_specs=[pl.BlockSpec((BM3, D), lambda i: (i, 0)),
                  pl.BlockSpec((BM3, D), lambda i: (i % (NP // BM3), 0)),
                  _full((1, D)), _full((1, D)), _full((D, D)), _full((1, D))],
        out_specs=pl.BlockSpec((BM3 // BM, BM), lambda i: (i, 0)),
        out_shape=jax.ShapeDtypeStruct((2 * NP // BM, BM), jnp.float32),
    )(agg, indeg, b, a, mW, mb)
    return out.reshape(-1)


def kernel(x, edge_index, W1, b1, W2, b2, prelu_a, mlp_W, mlp_b, batch_train):
    src = edge_index[0].astype(jnp.int32)
    dst = edge_index[1].astype(jnp.int32)
    pad = jnp.full((EP - E,), DUMMY, jnp.int32)
    srcp = jnp.concatenate([src, pad])
    dstp = jnp.concatenate([dst, pad])
    edges2 = jnp.stack([srcp, dstp]).reshape(2, ER, 128)
    src3 = jnp.stack([srcp, srcp + NP]).reshape(2, EP // 16, 16)
    dst2 = dstp.reshape(ER, 128)

    xpad = jnp.pad(x, ((0, NP - N), (0, 0)))
    perm = jax.random.permutation(jax.random.key(42), N).astype(jnp.int32)
    perm2 = jnp.concatenate([perm, jnp.zeros((NP - N,), jnp.int32)])
    perm2 = perm2.reshape(32, 5, 64)

    ones128 = jnp.ones((128, D), jnp.float32)
    zrows = jnp.zeros((RPT, D), jnp.float32)

    x2, outdeg, indeg = _get_sc_prep()(xpad, perm2, edges2, ones128, zrows)

    h1 = _stage1(x2, outdeg, W1)
    a1 = _get_sc_scatter()(h1, src3, dst2, zrows)
    h2 = _stage2(a1, indeg, outdeg, b1.reshape(1, D), prelu_a.reshape(1, D), W2)
    a2 = _get_sc_scatter()(h2, src3, dst2, zrows)
    lsum = _stage3(a2, indeg, b2.reshape(1, D), prelu_a.reshape(1, D),
                   mlp_W.T, mlp_b.reshape(1, D))

    return jnp.concatenate([lsum[:N], lsum[NP:NP + N]])


# speedup vs baseline: 1.1489x; 1.1489x over previous
import functools

import jax
import jax.numpy as jnp
from jax import lax
from jax.experimental import pallas as pl
from jax.experimental.pallas import tpu as pltpu
from jax.experimental.pallas import tpu_sc as plsc

N = 10000
NP = 10240
D = 128
E = 320000
NC, NS = 2, 16
CHT = 160
EP = CHT * NS * 128
ER = EP // 128
RPT = NP // NS
DUMMY = N

def _sc_prep_body(xpad, perm2, edges2, ones128, zrows,
                  x2_out, outdeg_out, indeg_out,
                  degacc, idx_v, ones_v, pidx_v, xbuf, sd):
    c = lax.axis_index("c")
    s = lax.axis_index("s")
    w = c * NS + s

    pltpu.sync_copy(zrows, degacc.at[pl.ds(s * RPT, RPT)])
    plsc.subcore_barrier()

    pltpu.sync_copy(ones128, ones_v)

    def deg_batch(b, carry):
        pltpu.sync_copy(edges2.at[c, pl.ds(s * CHT + b * IB, IB)], idx_v)

        def fire(j, carry2):
            pltpu.async_copy(ones_v, degacc.at[idx_v.at[j]], sd, add=True)
            return carry2
        lax.fori_loop(0, IB, fire, 0)

        def drain(j, carry2):
            pltpu.make_async_copy(ones_v, degacc.at[idx_v.at[0]], sd).wait()
            return carry2
        return lax.fori_loop(0, IB, drain, carry)
    lax.fori_loop(0, NB, deg_batch, 0)

    pltpu.sync_copy(xpad.at[pl.ds(w * (NP // 32), NP // 32)],
                    x2_out.at[pl.ds(w * (NP // 32), NP // 32)])
    pltpu.sync_copy(perm2.at[w], pidx_v)

    def xp_body(k, carry):
        chunk = w * 5 + k
        pltpu.sync_copy(xpad.at[pidx_v.at[k]], xbuf)
        pltpu.sync_copy(xbuf, x2_out.at[pl.ds(NP + chunk * 64, 64)])
        return carry
    lax.fori_loop(0, 5, xp_body, 0)

    plsc.subcore_barrier()

    @pl.when(c == 0)
    def _():
        pltpu.sync_copy(degacc.at[pl.ds(s * RPT, RPT)],
                        outdeg_out.at[pl.ds(s * RPT, RPT)])

    @pl.when(c == 1)
    def _():
        pltpu.sync_copy(degacc.at[pl.ds(s * RPT, RPT)],
                        indeg_out.at[pl.ds(s * RPT, RPT)])


@functools.cache
def _get_sc_prep():
    mesh = plsc.VectorSubcoreMesh(
        core_axis_name="c", subcore_axis_name="s",
        num_cores=NC, num_subcores=NS)
    return pl.kernel(
        _sc_prep_body,
        out_type=[
            jax.ShapeDtypeStruct((2 * NP, D), jnp.float32),
            jax.ShapeDtypeStruct((NP, D), jnp.float32),
            jax.ShapeDtypeStruct((NP, D), jnp.float32),
        ],
        mesh=mesh,
        scratch_types=[
            pltpu.VMEM_SHARED((NP, D), jnp.float32),
            pltpu.VMEM((IB, 128), jnp.int32),
            pltpu.VMEM((128, D), jnp.float32),
            pltpu.VMEM((5, 64), jnp.int32),
            pltpu.VMEM((64, D), jnp.float32),
            pltpu.SemaphoreType.DMA,
        ],
    )


IB = 16
NB = CHT // IB


def _sc_scatter_body(table, src3, dst2, zrows, agg_out,
                     acc, src_v, dst_v, buf0, buf1,
                     g0, g1, g2, g3, g4, g5, g6, g7, ss0, ss1):
    sgs0 = [g0, g1, g2, g3]
    sgs1 = [g4, g5, g6, g7]
    c = lax.axis_index("c")
    s = lax.axis_index("s")

    pltpu.sync_copy(zrows, acc.at[pl.ds(s * RPT, RPT)])
    plsc.subcore_barrier()

    def gath(g, q, buf, sem):
        pltpu.async_copy(table.at[src_v.at[g]],
                         buf.at[pl.ds(q * 32, 32)], sem)

    def wait_gath(q, buf, sem):
        pltpu.make_async_copy(table.at[src_v.at[0]],
                              buf.at[pl.ds(q * 32, 32)], sem).wait()

    def scat(m, buf, sem):
        pltpu.async_copy(buf, acc.at[dst_v.at[m]], sem, add=True)

    def wait_scat(buf, sem):
        pltpu.make_async_copy(buf, acc.at[dst_v.at[0]], sem).wait()

    def batch(b, carry):
        pltpu.sync_copy(src3.at[c, pl.ds(s * 4 * CHT + b * 4 * IB, 4 * IB)],
                        src_v)
        pltpu.sync_copy(dst2.at[pl.ds(s * CHT + b * IB, IB)], dst_v)
        for q in range(4):
            gath(q, q, buf0, sgs0[q])
            gath(4 + q, q, buf1, sgs1[q])

        def body(k, carry2):
            m = 2 * k
            for q in range(4):
                wait_gath(q, buf0, sgs0[q])
            scat(m, buf0, ss0)
            for q in range(4):
                wait_gath(q, buf1, sgs1[q])
            scat(m + 1, buf1, ss1)

            @pl.when(m + 2 < IB)
            def _():
                wait_scat(buf0, ss0)
                for q in range(4):
                    gath(4 * (m + 2) + q, q, buf0, sgs0[q])

            @pl.when(m + 3 < IB)
            def _():
                wait_scat(buf1, ss1)
                for q in range(4):
                    gath(4 * (m + 3) + q, q, buf1, sgs1[q])
            return carry2
        r = lax.fori_loop(0, IB // 2, body, carry)
        wait_scat(buf0, ss0)
        wait_scat(buf1, ss1)
        return r
    lax.fori_loop(0, NB, batch, 0)

    plsc.subcore_barrier()
    pltpu.sync_copy(acc.at[pl.ds(s * RPT, RPT)],
                    agg_out.at[pl.ds(c * NP + s * RPT, RPT)])


@functools.cache
def _get_sc_scatter():
    mesh = plsc.VectorSubcoreMesh(
        core_axis_name="c", subcore_axis_name="s",
        num_cores=NC, num_subcores=NS)
    return pl.kernel(
        _sc_scatter_body,
        out_type=jax.ShapeDtypeStruct((2 * NP, D), jnp.float32),
        mesh=mesh,
        scratch_types=[
            pltpu.VMEM_SHARED((NP, D), jnp.float32),
            pltpu.VMEM((4 * IB, 32), jnp.int32),
            pltpu.VMEM((IB, 128), jnp.int32),
            pltpu.VMEM((128, D), jnp.float32),
            pltpu.VMEM((128, D), jnp.float32),
        ] + [pltpu.SemaphoreType.DMA] * 10,
    )


BM = 256
GRID = (2 * NP) // BM


def _tc_stage1(x_ref, od_ref, w_ref, o_ref):
    ns = lax.rsqrt(jnp.maximum(od_ref[:, 0:1], 1.0))
    o_ref[...] = jnp.dot(x_ref[...] * ns, w_ref[...],
                         preferred_element_type=jnp.float32)


def _tc_stage2(agg_ref, id_ref, od_ref, b_ref, a_ref, w_ref, o_ref):
    nd = lax.rsqrt(jnp.maximum(id_ref[:, 0:1], 1.0))
    t = agg_ref[...] * nd + b_ref[...]
    p = jnp.maximum(t, 0.0) + a_ref[...] * jnp.minimum(t, 0.0)
    ns = lax.rsqrt(jnp.maximum(od_ref[:, 0:1], 1.0))
    o_ref[...] = jnp.dot(p * ns, w_ref[...],
                         preferred_element_type=jnp.float32)


BM3 = 2048


def _tc_stage3(agg_ref, id_ref, b_ref, a_ref, mwt_ref, mb_ref, o_ref):
    nd = lax.rsqrt(jnp.maximum(id_ref[:, 0:1], 1.0))
    t = agg_ref[...] * nd + b_ref[...]
    p = jnp.maximum(t, 0.0) + a_ref[...] * jnp.minimum(t, 0.0)
    q = jnp.dot(p, mwt_ref[...], preferred_element_type=jnp.float32)
    q = q + mb_ref[...]
    o_ref[...] = jnp.sum(q, axis=1).reshape(BM3 // BM, BM)


def _row_spec():
    return pl.BlockSpec((BM, D), lambda i: (i, 0))


def _deg_spec():
    return pl.BlockSpec((BM, D), lambda i: (i % (NP // BM), 0))


def _full(shape):
    return pl.BlockSpec(shape, lambda i: tuple(0 for _ in shape))


def _stage1(x2, outdeg, W1):
    return pl.pallas_call(
        _tc_stage1, grid=(GRID,),
        in_specs=[_row_spec(), _deg_spec(), _full((D, D))],
        out_specs=_row_spec(),
        out_shape=jax.ShapeDtypeStruct((2 * NP, D), jnp.float32),
    )(x2, outdeg, W1)


def _stage2(agg, indeg, outdeg, b, a, W2):
    return pl.pallas_call(
        _tc_stage2, grid=(GRID,),
        in_specs=[_row_spec(), _deg_spec(), _deg_spec(),
                  _full((1, D)), _full((1, D)), _full((D, D))],
        out_specs=_row_spec(),
        out_shape=jax.ShapeDtypeStruct((2 * NP, D), jnp.float32),
    )(agg, indeg, outdeg, b, a, W2)


def _stage3(agg, indeg, b, a, mW, mb):
    grid3 = (2 * NP) // BM3
    out = pl.pallas_call(
        _tc_stage3, grid=(grid3,),
        in_specs=[pl.BlockSpec((BM3, D), lambda i: (i, 0)),
                  pl.BlockSpec((BM3, D), lambda i: (i % (NP // BM3), 0)),
                  _full((1, D)), _full((1, D)), _full((D, D)), _full((1, D))],
        out_specs=pl.BlockSpec((BM3 // BM, BM), lambda i: (i, 0)),
        out_shape=jax.ShapeDtypeStruct((2 * NP // BM, BM), jnp.float32),
    )(agg, indeg, b, a, mW, mb)
    return out.reshape(-1)


def kernel(x, edge_index, W1, b1, W2, b2, prelu_a, mlp_W, mlp_b, batch_train):
    src = edge_index[0].astype(jnp.int32)
    dst = edge_index[1].astype(jnp.int32)
    pad = jnp.full((EP - E,), DUMMY, jnp.int32)
    srcp = jnp.concatenate([src, pad])
    dstp = jnp.concatenate([dst, pad])
    edges2 = jnp.stack([srcp, dstp]).reshape(2, ER, 128)
    src3 = jnp.stack([srcp, srcp + NP]).reshape(2, EP // 32, 32)
    dst2 = dstp.reshape(ER, 128)

    xpad = jnp.pad(x, ((0, NP - N), (0, 0)))
    perm = jax.random.permutation(jax.random.key(42), N).astype(jnp.int32)
    perm2 = jnp.concatenate([perm, jnp.zeros((NP - N,), jnp.int32)])
    perm2 = perm2.reshape(32, 5, 64)

    ones128 = jnp.ones((128, D), jnp.float32)
    zrows = jnp.zeros((RPT, D), jnp.float32)

    x2, outdeg, indeg = _get_sc_prep()(xpad, perm2, edges2, ones128, zrows)

    h1 = _stage1(x2, outdeg, W1)
    a1 = _get_sc_scatter()(h1, src3, dst2, zrows)
    h2 = _stage2(a1, indeg, outdeg, b1.reshape(1, D), prelu_a.reshape(1, D), W2)
    a2 = _get_sc_scatter()(h2, src3, dst2, zrows)
    lsum = _stage3(a2, indeg, b2.reshape(1, D), prelu_a.reshape(1, D),
                   mlp_W.T, mlp_b.reshape(1, D))

    return jnp.concatenate([lsum[:N], lsum[NP:NP + N]])
